# Initial kernel scaffold; baseline (speedup 1.0000x reference)
import jax
import jax.numpy as jnp
from jax.experimental import pallas as pl


def kernel(x, weight, lora_A, lora_B, router_h, router_d, limes):
    raise NotImplementedError("write your pallas kernel here")


# speedup vs baseline: 3.3359x; 3.3359x over previous
import functools

import jax
import jax.numpy as jnp
from jax import lax
from jax.experimental import pallas as pl
from jax.experimental.pallas import tpu as pltpu

IN_F = 768
OUT_F = 768
R = 8
E = 64
TOPK = 8
NGRAM = 2
GAMMA = 0.5
TEMP = 1.0
ALPHA = 16.0

AB = 1024
HB = 1024
TB = 1024


def _anchor_kernel(x3_ref, la_ref, rh_ref, rd_ref, h_ref, ds_ref, hs_ref, dss_ref):
    j = pl.program_id(0)
    xa = x3_ref[...]
    h = lax.dot_general(xa, rh_ref[...], (((1,), (0,)), ((), ())),
                        preferred_element_type=jnp.float32)
    da = lax.dot_general(xa, la_ref[...], (((1,), (1,)), ((), ())),
                         preferred_element_type=jnp.float32)
    ds = lax.dot_general(da, rd_ref[...], (((1,), (0,)), ((), ())),
                         preferred_element_type=jnp.float32)
    h_ref[...] = h
    ds_ref[...] = ds
    hmax = jnp.max(jnp.abs(h), keepdims=True).reshape(1, 1)
    dmax = jnp.max(jnp.abs(ds), keepdims=True).reshape(1, 1)

    @pl.when(j == 0)
    def _():
        hs_ref[...] = hmax
        dss_ref[...] = dmax

    @pl.when(j > 0)
    def _():
        hs_ref[...] = jnp.maximum(hs_ref[...], hmax)
        dss_ref[...] = jnp.maximum(dss_ref[...], dmax)


def _routing_kernel(h_ref, ds_ref, hs_ref, dss_ref, limes_ref, p_ref):
    eps = 1e-6
    c1 = (1.0 - GAMMA) / jnp.maximum(hs_ref[...], eps)
    c2 = GAMMA / jnp.maximum(dss_ref[...], eps)
    logits = (c1 * h_ref[...] + c2 * ds_ref[...]) * (1.0 / max(TEMP, eps))
    m = jnp.max(logits, axis=-1, keepdims=True)
    z = jnp.exp(logits - m)
    lane = lax.broadcasted_iota(jnp.int32, (HB, E), 1)
    act = z
    sel = jnp.zeros((HB, E), dtype=jnp.float32)
    for _ in range(TOPK):
        mx = jnp.max(act, axis=-1, keepdims=True)
        cand = act == mx
        first = jnp.min(jnp.where(cand, lane, E), axis=-1, keepdims=True)
        pick = lane == first
        sel = jnp.where(pick, 1.0, sel)
        act = jnp.where(pick, -jnp.inf, act)
    w = z * sel
    s = jnp.maximum(jnp.sum(w, axis=-1, keepdims=True), 1e-9)
    mix = lax.dot_general(w, limes_ref[...], (((1,), (0,)), ((), ())),
                          preferred_element_type=jnp.float32)
    p_ref[...] = mix / s


def _out_kernel(xf_ref, w_ref, la_ref, lb_ref, p_ref, o_ref):
    xb = xf_ref[...]
    xw = lax.dot_general(xb, w_ref[...], (((1,), (1,)), ((), ())),
                         preferred_element_type=jnp.float32)
    delta = lax.dot_general(xb, la_ref[...], (((1,), (1,)), ((), ())),
                            preferred_element_type=jnp.float32)
    hbc = TB // NGRAM
    rows = lax.broadcasted_iota(jnp.int32, (TB, hbc), 0) // NGRAM
    cols = lax.broadcasted_iota(jnp.int32, (TB, hbc), 1)
    erep = (rows == cols).astype(jnp.float32)
    p_full = lax.dot_general(erep, p_ref[...], (((1,), (0,)), ((), ())),
                             preferred_element_type=jnp.float32)
    q = delta * p_full
    lora = lax.dot_general(q, lb_ref[...], (((1,), (1,)), ((), ())),
                           preferred_element_type=jnp.float32)
    o_ref[...] = xw + lora * (ALPHA / R)


def kernel(x, weight, lora_A, lora_B, router_h, router_d, limes):
    Bsz, T, _ = x.shape
    na = (T // NGRAM) * Bsz
    bt = Bsz * T
    x3 = x.reshape(na, NGRAM * IN_F)
    xf = x.reshape(bt, IN_F)

    h_all, ds_all, hs, dss = pl.pallas_call(
        _anchor_kernel,
        grid=(na // AB,),
        in_specs=[
            pl.BlockSpec((AB, IN_F), lambda j: (j, NGRAM - 1)),
            pl.BlockSpec((R, IN_F), lambda j: (0, 0)),
            pl.BlockSpec((IN_F, E), lambda j: (0, 0)),
            pl.BlockSpec((R, E), lambda j: (0, 0)),
        ],
        out_specs=[
            pl.BlockSpec((AB, E), lambda j: (j, 0)),
            pl.BlockSpec((AB, E), lambda j: (j, 0)),
            pl.BlockSpec((1, 1), lambda j: (0, 0)),
            pl.BlockSpec((1, 1), lambda j: (0, 0)),
        ],
        out_shape=[
            jax.ShapeDtypeStruct((na, E), jnp.float32),
            jax.ShapeDtypeStruct((na, E), jnp.float32),
            jax.ShapeDtypeStruct((1, 1), jnp.float32),
            jax.ShapeDtypeStruct((1, 1), jnp.float32),
        ],
    )(x3, lora_A, router_h, router_d)

    p_mix = pl.pallas_call(
        _routing_kernel,
        grid=(na // HB,),
        in_specs=[
            pl.BlockSpec((HB, E), lambda j: (j, 0)),
            pl.BlockSpec((HB, E), lambda j: (j, 0)),
            pl.BlockSpec((1, 1), lambda j: (0, 0)),
            pl.BlockSpec((1, 1), lambda j: (0, 0)),
            pl.BlockSpec((E, R), lambda j: (0, 0)),
        ],
        out_specs=pl.BlockSpec((HB, R), lambda j: (j, 0)),
        out_shape=jax.ShapeDtypeStruct((na, R), jnp.float32),
    )(h_all, ds_all, hs, dss, limes)

    out = pl.pallas_call(
        _out_kernel,
        grid=(bt // TB,),
        in_specs=[
            pl.BlockSpec((TB, IN_F), lambda j: (j, 0)),
            pl.BlockSpec((OUT_F, IN_F), lambda j: (0, 0)),
            pl.BlockSpec((R, IN_F), lambda j: (0, 0)),
            pl.BlockSpec((OUT_F, R), lambda j: (0, 0)),
            pl.BlockSpec((TB // NGRAM, R), lambda j: (j, 0)),
        ],
        out_specs=pl.BlockSpec((TB, OUT_F), lambda j: (j, 0)),
        out_shape=jax.ShapeDtypeStruct((bt, OUT_F), jnp.float32),
    )(xf, weight, lora_A, lora_B, p_mix)

    return out.reshape(Bsz, T, OUT_F)


# speedup vs baseline: 3.3513x; 1.0046x over previous
import functools

import jax
import jax.numpy as jnp
from jax import lax
from jax.experimental import pallas as pl
from jax.experimental.pallas import tpu as pltpu

IN_F = 768
OUT_F = 768
R = 8
E = 64
TOPK = 8
NGRAM = 2
GAMMA = 0.5
TEMP = 1.0
ALPHA = 16.0

AB = 1024
HB = 1024
TB = 1024


def _anchor_kernel(x3_ref, la_ref, rh_ref, rd_ref, h_ref, ds_ref, hs_ref, dss_ref):
    j = pl.program_id(0)
    xa = x3_ref[...]
    h = lax.dot_general(xa, rh_ref[...], (((1,), (0,)), ((), ())),
                        preferred_element_type=jnp.float32)
    da = lax.dot_general(xa, la_ref[...], (((1,), (1,)), ((), ())),
                         preferred_element_type=jnp.float32)
    ds = lax.dot_general(da, rd_ref[...], (((1,), (0,)), ((), ())),
                         preferred_element_type=jnp.float32)
    h_ref[...] = h
    ds_ref[...] = ds
    hmax = jnp.max(jnp.abs(h), keepdims=True).reshape(1, 1)
    dmax = jnp.max(jnp.abs(ds), keepdims=True).reshape(1, 1)

    @pl.when(j == 0)
    def _():
        hs_ref[...] = hmax
        dss_ref[...] = dmax

    @pl.when(j > 0)
    def _():
        hs_ref[...] = jnp.maximum(hs_ref[...], hmax)
        dss_ref[...] = jnp.maximum(dss_ref[...], dmax)


def _routing_kernel(h_ref, ds_ref, hs_ref, dss_ref, limes_ref, p_ref):
    eps = 1e-6
    c1 = (1.0 - GAMMA) / jnp.maximum(hs_ref[...], eps)
    c2 = GAMMA / jnp.maximum(dss_ref[...], eps)
    logits = (c1 * h_ref[...] + c2 * ds_ref[...]) * (1.0 / max(TEMP, eps))
    m = jnp.max(logits, axis=-1, keepdims=True)
    z = jnp.exp(logits - m)
    lane = lax.broadcasted_iota(jnp.int32, (HB, E), 1)
    act = z
    sel = jnp.zeros((HB, E), dtype=jnp.float32)
    for _ in range(TOPK):
        mx = jnp.max(act, axis=-1, keepdims=True)
        cand = act == mx
        first = jnp.min(jnp.where(cand, lane, E), axis=-1, keepdims=True)
        pick = lane == first
        sel = jnp.where(pick, 1.0, sel)
        act = jnp.where(pick, -jnp.inf, act)
    w = z * sel
    s = jnp.maximum(jnp.sum(w, axis=-1, keepdims=True), 1e-9)
    mix = lax.dot_general(w, limes_ref[...], (((1,), (0,)), ((), ())),
                          preferred_element_type=jnp.float32)
    p_ref[...] = mix / s


def _out_kernel(xf_ref, w_ref, la_ref, lb_ref, p_ref, o_ref):
    xb = xf_ref[...]
    xw = lax.dot_general(xb.astype(jnp.bfloat16), w_ref[...].astype(jnp.bfloat16),
                         (((1,), (1,)), ((), ())),
                         preferred_element_type=jnp.float32)
    delta = lax.dot_general(xb, la_ref[...], (((1,), (1,)), ((), ())),
                            preferred_element_type=jnp.float32)
    hbc = TB // NGRAM
    rows = lax.broadcasted_iota(jnp.int32, (TB, hbc), 0) // NGRAM
    cols = lax.broadcasted_iota(jnp.int32, (TB, hbc), 1)
    erep = (rows == cols).astype(jnp.float32)
    p_full = lax.dot_general(erep, p_ref[...], (((1,), (0,)), ((), ())),
                             preferred_element_type=jnp.float32)
    q = delta * p_full
    lora = lax.dot_general(q, lb_ref[...], (((1,), (1,)), ((), ())),
                           preferred_element_type=jnp.float32)
    o_ref[...] = xw + lora * (ALPHA / R)


def kernel(x, weight, lora_A, lora_B, router_h, router_d, limes):
    Bsz, T, _ = x.shape
    na = (T // NGRAM) * Bsz
    bt = Bsz * T
    x3 = x.reshape(na, NGRAM * IN_F)
    xf = x.reshape(bt, IN_F)

    h_all, ds_all, hs, dss = pl.pallas_call(
        _anchor_kernel,
        grid=(na // AB,),
        in_specs=[
            pl.BlockSpec((AB, IN_F), lambda j: (j, NGRAM - 1)),
            pl.BlockSpec((R, IN_F), lambda j: (0, 0)),
            pl.BlockSpec((IN_F, E), lambda j: (0, 0)),
            pl.BlockSpec((R, E), lambda j: (0, 0)),
        ],
        out_specs=[
            pl.BlockSpec((AB, E), lambda j: (j, 0)),
            pl.BlockSpec((AB, E), lambda j: (j, 0)),
            pl.BlockSpec((1, 1), lambda j: (0, 0)),
            pl.BlockSpec((1, 1), lambda j: (0, 0)),
        ],
        out_shape=[
            jax.ShapeDtypeStruct((na, E), jnp.float32),
            jax.ShapeDtypeStruct((na, E), jnp.float32),
            jax.ShapeDtypeStruct((1, 1), jnp.float32),
            jax.ShapeDtypeStruct((1, 1), jnp.float32),
        ],
    )(x3, lora_A, router_h, router_d)

    p_mix = pl.pallas_call(
        _routing_kernel,
        grid=(na // HB,),
        in_specs=[
            pl.BlockSpec((HB, E), lambda j: (j, 0)),
            pl.BlockSpec((HB, E), lambda j: (j, 0)),
            pl.BlockSpec((1, 1), lambda j: (0, 0)),
            pl.BlockSpec((1, 1), lambda j: (0, 0)),
            pl.BlockSpec((E, R), lambda j: (0, 0)),
        ],
        out_specs=pl.BlockSpec((HB, R), lambda j: (j, 0)),
        out_shape=jax.ShapeDtypeStruct((na, R), jnp.float32),
    )(h_all, ds_all, hs, dss, limes)

    out = pl.pallas_call(
        _out_kernel,
        grid=(bt // TB,),
        in_specs=[
            pl.BlockSpec((TB, IN_F), lambda j: (j, 0)),
            pl.BlockSpec((OUT_F, IN_F), lambda j: (0, 0)),
            pl.BlockSpec((R, IN_F), lambda j: (0, 0)),
            pl.BlockSpec((OUT_F, R), lambda j: (0, 0)),
            pl.BlockSpec((TB // NGRAM, R), lambda j: (j, 0)),
        ],
        out_specs=pl.BlockSpec((TB, OUT_F), lambda j: (j, 0)),
        out_shape=jax.ShapeDtypeStruct((bt, OUT_F), jnp.float32),
    )(xf, weight, lora_A, lora_B, p_mix)

    return out.reshape(Bsz, T, OUT_F)


# speedup vs baseline: 3.5376x; 1.0556x over previous
import functools

import jax
import jax.numpy as jnp
from jax import lax
from jax.experimental import pallas as pl
from jax.experimental.pallas import tpu as pltpu

IN_F = 768
OUT_F = 768
R = 8
E = 64
TOPK = 8
NGRAM = 2
GAMMA = 0.5
TEMP = 1.0
ALPHA = 16.0

AB = 1024
HB = 1024
TB = 1024


def _anchor_kernel(x3_ref, la_ref, rh_ref, rd_ref, h_ref, ds_ref, hs_ref, dss_ref):
    j = pl.program_id(0)
    xa = x3_ref[...]
    h = lax.dot_general(xa, rh_ref[...], (((1,), (0,)), ((), ())),
                        preferred_element_type=jnp.float32)
    da = lax.dot_general(xa, la_ref[...], (((1,), (1,)), ((), ())),
                         preferred_element_type=jnp.float32)
    ds = lax.dot_general(da, rd_ref[...], (((1,), (0,)), ((), ())),
                         preferred_element_type=jnp.float32)
    h_ref[...] = h
    ds_ref[...] = ds
    hmax = jnp.max(jnp.abs(h), keepdims=True).reshape(1, 1)
    dmax = jnp.max(jnp.abs(ds), keepdims=True).reshape(1, 1)

    @pl.when(j == 0)
    def _():
        hs_ref[...] = hmax
        dss_ref[...] = dmax

    @pl.when(j > 0)
    def _():
        hs_ref[...] = jnp.maximum(hs_ref[...], hmax)
        dss_ref[...] = jnp.maximum(dss_ref[...], dmax)


def _routing_kernel(h_ref, ds_ref, hs_ref, dss_ref, limes_ref, p_ref):
    eps = 1e-6
    c1 = (1.0 - GAMMA) / jnp.maximum(hs_ref[...], eps)
    c2 = GAMMA / jnp.maximum(dss_ref[...], eps)
    logits = (c1 * h_ref[...] + c2 * ds_ref[...]) * (1.0 / max(TEMP, eps))
    m = jnp.max(logits, axis=-1, keepdims=True)
    z = jnp.exp(logits - m)
    lane = lax.broadcasted_iota(jnp.int32, (HB, E), 1)
    bits = lax.bitcast_convert_type(z, jnp.int32)
    act = (bits & ~jnp.int32(E - 1)) | (jnp.int32(E - 1) - lane)
    sel = jnp.zeros((HB, E), dtype=jnp.float32)
    for _ in range(TOPK):
        mx = jnp.max(act, axis=-1, keepdims=True)
        pick = act == mx
        sel = jnp.where(pick, 1.0, sel)
        act = jnp.where(pick, jnp.int32(-2147483648), act)
    w = z * sel
    s = jnp.maximum(jnp.sum(w, axis=-1, keepdims=True), 1e-9)
    mix = lax.dot_general(w, limes_ref[...], (((1,), (0,)), ((), ())),
                          preferred_element_type=jnp.float32)
    p_ref[...] = mix / s


def _out_kernel(xf_ref, w_ref, la_ref, lb_ref, p_ref, o_ref):
    xb = xf_ref[...]
    xw = lax.dot_general(xb.astype(jnp.bfloat16), w_ref[...].astype(jnp.bfloat16),
                         (((1,), (1,)), ((), ())),
                         preferred_element_type=jnp.float32)
    delta = lax.dot_general(xb, la_ref[...], (((1,), (1,)), ((), ())),
                            preferred_element_type=jnp.float32)
    hbc = TB // NGRAM
    rows = lax.broadcasted_iota(jnp.int32, (TB, hbc), 0) // NGRAM
    cols = lax.broadcasted_iota(jnp.int32, (TB, hbc), 1)
    erep = (rows == cols).astype(jnp.float32)
    p_full = lax.dot_general(erep, p_ref[...], (((1,), (0,)), ((), ())),
                             preferred_element_type=jnp.float32)
    q = delta * p_full
    lora = lax.dot_general(q, lb_ref[...], (((1,), (1,)), ((), ())),
                           preferred_element_type=jnp.float32)
    o_ref[...] = xw + lora * (ALPHA / R)


def kernel(x, weight, lora_A, lora_B, router_h, router_d, limes):
    Bsz, T, _ = x.shape
    na = (T // NGRAM) * Bsz
    bt = Bsz * T
    x3 = x.reshape(na, NGRAM * IN_F)
    xf = x.reshape(bt, IN_F)

    h_all, ds_all, hs, dss = pl.pallas_call(
        _anchor_kernel,
        grid=(na // AB,),
        in_specs=[
            pl.BlockSpec((AB, IN_F), lambda j: (j, NGRAM - 1)),
            pl.BlockSpec((R, IN_F), lambda j: (0, 0)),
            pl.BlockSpec((IN_F, E), lambda j: (0, 0)),
            pl.BlockSpec((R, E), lambda j: (0, 0)),
        ],
        out_specs=[
            pl.BlockSpec((AB, E), lambda j: (j, 0)),
            pl.BlockSpec((AB, E), lambda j: (j, 0)),
            pl.BlockSpec((1, 1), lambda j: (0, 0)),
            pl.BlockSpec((1, 1), lambda j: (0, 0)),
        ],
        out_shape=[
            jax.ShapeDtypeStruct((na, E), jnp.float32),
            jax.ShapeDtypeStruct((na, E), jnp.float32),
            jax.ShapeDtypeStruct((1, 1), jnp.float32),
            jax.ShapeDtypeStruct((1, 1), jnp.float32),
        ],
    )(x3, lora_A, router_h, router_d)

    p_mix = pl.pallas_call(
        _routing_kernel,
        grid=(na // HB,),
        in_specs=[
            pl.BlockSpec((HB, E), lambda j: (j, 0)),
            pl.BlockSpec((HB, E), lambda j: (j, 0)),
            pl.BlockSpec((1, 1), lambda j: (0, 0)),
            pl.BlockSpec((1, 1), lambda j: (0, 0)),
            pl.BlockSpec((E, R), lambda j: (0, 0)),
        ],
        out_specs=pl.BlockSpec((HB, R), lambda j: (j, 0)),
        out_shape=jax.ShapeDtypeStruct((na, R), jnp.float32),
    )(h_all, ds_all, hs, dss, limes)

    out = pl.pallas_call(
        _out_kernel,
        grid=(bt // TB,),
        in_specs=[
            pl.BlockSpec((TB, IN_F), lambda j: (j, 0)),
            pl.BlockSpec((OUT_F, IN_F), lambda j: (0, 0)),
            pl.BlockSpec((R, IN_F), lambda j: (0, 0)),
            pl.BlockSpec((OUT_F, R), lambda j: (0, 0)),
            pl.BlockSpec((TB // NGRAM, R), lambda j: (j, 0)),
        ],
        out_specs=pl.BlockSpec((TB, OUT_F), lambda j: (j, 0)),
        out_shape=jax.ShapeDtypeStruct((bt, OUT_F), jnp.float32),
    )(xf, weight, lora_A, lora_B, p_mix)

    return out.reshape(Bsz, T, OUT_F)
